# Initial kernel scaffold; baseline (speedup 1.0000x reference)
import functools

import jax
import jax.numpy as jnp
from jax import lax
from jax.experimental import pallas as pl
from jax.experimental.pallas import tpu as pltpu
from jax.experimental.pallas import tpu_sc as plsc

BATCH = 16384
SEQ = 200
D = 100
B = BATCH * SEQ

NC = 2
NS = 16
NW = NC * NS

G = 128
K = 8
CHUNK = K * G
ROWS = B // G
ROWS_PER_W = ROWS // NW
N_CHUNKS = ROWS_PER_W // K


def _sc_gather(off2, table):
    mesh = plsc.VectorSubcoreMesh(core_axis_name="c", subcore_axis_name="s")

    @functools.partial(
        pl.kernel,
        mesh=mesh,
        out_type=jax.ShapeDtypeStruct((ROWS, G, D), jnp.float32),
        scratch_types=[
            pltpu.VMEM((K, G), jnp.int32),
            pltpu.VMEM((K, G, D), jnp.float32),
            pltpu.SemaphoreType.DMA,
        ],
    )
    def k(off_hbm, tab_hbm, out_hbm, idx_v, rows_v, sem):
        wid = lax.axis_index("s") * NC + lax.axis_index("c")
        row0 = wid * ROWS_PER_W

        def chunk(i, carry):
            r = row0 + i * K
            pltpu.sync_copy(off_hbm.at[pl.ds(r, K)], idx_v)
            for j in range(K):
                pltpu.async_copy(tab_hbm.at[idx_v.at[j]], rows_v.at[j], sem)
            for j in range(K):
                pltpu.make_async_copy(
                    tab_hbm.at[idx_v.at[j]], rows_v.at[j], sem).wait()
            pltpu.sync_copy(rows_v, out_hbm.at[pl.ds(r, K)])
            return carry

        lax.fori_loop(0, N_CHUNKS, chunk, 0)

    return k(off2, table)


def kernel(offsets, offset_embedding):
    off2 = offsets.reshape(ROWS, G)
    out = _sc_gather(off2, offset_embedding)
    return out.reshape(BATCH, SEQ, D)


# speedup vs baseline: 2.5971x; 2.5971x over previous
import functools

import jax
import jax.numpy as jnp
from jax import lax
from jax.experimental import pallas as pl
from jax.experimental.pallas import tpu as pltpu
from jax.experimental.pallas import tpu_sc as plsc

BATCH = 16384
SEQ = 200
D = 100
DP = 112
B = BATCH * SEQ

NC = 2
NS = 16
NW = NC * NS

G = 128
K = 8
ROWS = B // G
ROWS_PER_W = ROWS // NW
N_CHUNKS = ROWS_PER_W // K


def _sc_gather(off2, tab_padded):
    mesh = plsc.VectorSubcoreMesh(core_axis_name="c", subcore_axis_name="s")

    @functools.partial(
        pl.kernel,
        mesh=mesh,
        out_type=jax.ShapeDtypeStruct((ROWS, G, DP), jnp.float32),
        scratch_types=[
            pltpu.VMEM((K, G), jnp.int32),
            pltpu.VMEM((K, G, DP), jnp.float32),
            pltpu.SemaphoreType.DMA,
        ],
        compiler_params=pltpu.CompilerParams(use_tc_tiling_on_sc=False),
    )
    def k(off_hbm, tab_hbm, out_hbm, idx_v, rows_v, sem):
        wid = lax.axis_index("s") * NC + lax.axis_index("c")
        row0 = wid * ROWS_PER_W

        def chunk(i, carry):
            r = row0 + i * K
            pltpu.sync_copy(off_hbm.at[pl.ds(r, K)], idx_v)
            handles = [
                pltpu.async_copy(tab_hbm.at[idx_v.at[j]], rows_v.at[j], sem)
                for j in range(K)
            ]
            for h in handles:
                h.wait()
            pltpu.sync_copy(rows_v, out_hbm.at[pl.ds(r, K)])
            return carry

        lax.fori_loop(0, N_CHUNKS, chunk, 0)

    return k(off2, tab_padded)


def kernel(offsets, offset_embedding):
    off2 = offsets.reshape(ROWS, G)
    tab_padded = jnp.pad(offset_embedding, ((0, 0), (0, DP - D)))
    out = _sc_gather(off2, tab_padded)
    return out[:, :, :D].reshape(BATCH, SEQ, D)


# speedup vs baseline: 7.2934x; 2.8083x over previous
import functools

import jax
import jax.numpy as jnp
from jax import lax
from jax.experimental import pallas as pl
from jax.experimental.pallas import tpu as pltpu
from jax.experimental.pallas import tpu_sc as plsc

BATCH = 16384
SEQ = 200
D = 100
DP = 112
VOCAB = 400
B = BATCH * SEQ

NC = 2
NS = 16
NW = NC * NS

G = 128
K = 2
ROWS = B // G
ROWS_PER_W = ROWS // NW
N_CHUNKS = ROWS_PER_W // K


def _sc_gather(off2, tab_padded):
    mesh = plsc.VectorSubcoreMesh(core_axis_name="c", subcore_axis_name="s")

    @functools.partial(
        pl.kernel,
        mesh=mesh,
        out_type=jax.ShapeDtypeStruct((ROWS, G, DP), jnp.float32),
        scratch_types=[
            pltpu.VMEM_SHARED((VOCAB, DP), jnp.float32),
            pltpu.VMEM((2, K, G), jnp.int32),
            pltpu.VMEM((2, K, G, DP), jnp.float32),
            pltpu.SemaphoreType.DMA,
            pltpu.SemaphoreType.DMA,
            pltpu.SemaphoreType.DMA,
            pltpu.SemaphoreType.DMA,
        ],
    )
    def k(off_hbm, tab_hbm, out_hbm, tab_s, idx_v, rows_v, g0, g1, o0, o1):
        cid = lax.axis_index("c")
        sid = lax.axis_index("s")
        wid = sid * NC + cid
        row0 = wid * ROWS_PER_W
        gsem = (g0, g1)
        osem = (o0, o1)

        @pl.when(sid == 0)
        def _():
            pltpu.sync_copy(tab_hbm, tab_s)

        plsc.subcore_barrier()

        def fire(i, b):
            r = row0 + i * K
            pltpu.sync_copy(off_hbm.at[pl.ds(r, K)], idx_v.at[b])
            for j in range(K):
                pltpu.async_copy(
                    tab_s.at[idx_v.at[b].at[j]], rows_v.at[b].at[j], gsem[b])

        def drain_and_put(i, b):
            for j in range(K):
                pltpu.make_async_copy(
                    tab_s.at[idx_v.at[b].at[j]], rows_v.at[b].at[j],
                    gsem[b]).wait()
            r = row0 + i * K
            pltpu.async_copy(rows_v.at[b], out_hbm.at[pl.ds(r, K)], osem[b])

        def wait_out(i, b):
            r = row0 + i * K
            pltpu.make_async_copy(
                rows_v.at[b], out_hbm.at[pl.ds(r, K)], osem[b]).wait()

        def pair(p, carry):
            for b in range(2):
                i = p * 2 + b

                @pl.when(i >= 2)
                def _():
                    wait_out(i - 2, b)

                fire(i, b)

                @pl.when(i >= 1)
                def _():
                    drain_and_put(i - 1, 1 - b)

            return carry

        lax.fori_loop(0, N_CHUNKS // 2, pair, 0)
        drain_and_put(N_CHUNKS - 1, (N_CHUNKS - 1) % 2)
        wait_out(N_CHUNKS - 2, 0)
        wait_out(N_CHUNKS - 1, 1)

    return k(off2, tab_padded)


def kernel(offsets, offset_embedding):
    off2 = offsets.reshape(ROWS, G)
    tab_padded = jnp.pad(offset_embedding, ((0, 0), (0, DP - D)))
    out = _sc_gather(off2, tab_padded)
    return out[:, :, :D].reshape(BATCH, SEQ, D)
